# Initial kernel scaffold; baseline (speedup 1.0000x reference)
import jax
import jax.numpy as jnp
from jax.experimental import pallas as pl


def kernel(feat0, feat1, feat2, feat3, boxes0, boxes1):
    raise NotImplementedError("write your pallas kernel here")


# speedup vs baseline: 28.4566x; 28.4566x over previous
import functools

import jax
import jax.numpy as jnp
from jax import lax
from jax.experimental import pallas as pl
from jax.experimental.pallas import tpu as pltpu
from jax.experimental.pallas import tpu_sc as plsc

_OUT = 7
_NBINS = _OUT * _OUT
_SR = 16
_L = 16
_NC, _NS = 2, 16
_NW = _NC * _NS
_C = 256
_K = 256
_R_PER_W = _K // _NW
_CHUNK_BINS = 7
_CHUNK_ROWS = _CHUNK_BINS * _SR
_NCHUNKS = _NBINS // _CHUNK_BINS

_SCALES = (0.25, 0.125, 0.0625, 0.03125)
_HS = (128, 64, 32, 16)
_LVL_OFF = (0, 32768, 40960, 43008)
_CANONICAL_SCALE = 224.0
_CANONICAL_LEVEL = 4.0
_K_MIN, _K_MAX = 2.0, 5.0


def _make_mesh():
    return plsc.VectorSubcoreMesh(
        core_axis_name="c", subcore_axis_name="s",
        num_cores=_NC, num_subcores=_NS)


@functools.partial(
    pl.kernel,
    out_type=jax.ShapeDtypeStruct((_K, _C, _NBINS), jnp.float32),
    mesh=_make_mesh(),
    scratch_types=[
        pltpu.VMEM((2, _L), jnp.float32),
        pltpu.VMEM((_NCHUNKS, _CHUNK_ROWS), jnp.int32),
        pltpu.VMEM((_NBINS * _SR,), jnp.float32),
        pltpu.VMEM((_CHUNK_ROWS, _C), jnp.float32),
        pltpu.VMEM((_C, _NBINS), jnp.float32),
        pltpu.SemaphoreType.DMA,
    ],
    compiler_params=pltpu.CompilerParams(needs_layout_passes=False),
)
def _roi_align_sc(table, params, out, p_v, idx_v, w_v, rows_v, out_v, sem):
    wid = lax.axis_index("s") * _NC + lax.axis_index("c")
    lane = lax.iota(jnp.int32, _L)
    corner = lane & 3
    sub = lane >> 2
    offy = ((sub >> 1).astype(jnp.float32) + 0.5) * 0.5
    offx = ((sub & 1).astype(jnp.float32) + 0.5) * 0.5
    dy = (corner >> 1) == 1
    dx = (corner & 1) == 1

    def roi_body(i, _):
        r = wid * _R_PER_W + i
        pb = i & 1
        pltpu.sync_copy(params.at[r], p_v.at[pb])

        def splat(j):
            return plsc.load_gather(
                p_v, [jnp.broadcast_to(pb, (_L,)),
                      jnp.full((_L,), j, jnp.int32)])

        cw = splat(0)
        ch = splat(1)
        bin_h = splat(2)
        bin_w = splat(3)
        start_h = splat(4)
        start_w = splat(5)
        cos_t = splat(6)
        sin_t = splat(7)
        hf = splat(8)
        wf = splat(9)
        base_i = splat(10).astype(jnp.int32)
        w_stride = splat(11).astype(jnp.int32)
        hm1 = (hf - 1.0).astype(jnp.int32)
        wm1 = (wf - 1.0).astype(jnp.int32)

        def idx_gen(b, _):
            cb = (b * 9363) >> 16
            jb = b - cb * 7
            pyf = cb.astype(jnp.float32)
            pxf = jb.astype(jnp.float32)
            yy = start_h + (pyf + offy) * bin_h
            xx = start_w + (pxf + offx) * bin_w
            y = yy * cos_t - xx * sin_t + ch
            x = yy * sin_t + xx * cos_t + cw
            validf = jnp.where(
                (y > -1.0) & (y < hf) & (x > -1.0) & (x < wf), 0.25, 0.0)
            yc = jnp.clip(y, 0.0, hf - 1.0)
            xc = jnp.clip(x, 0.0, wf - 1.0)
            y0 = yc.astype(jnp.int32)
            x0 = xc.astype(jnp.int32)
            ly = yc - y0.astype(jnp.float32)
            lx = xc - x0.astype(jnp.float32)
            wgt = (jnp.where(dy, ly, 1.0 - ly)
                   * jnp.where(dx, lx, 1.0 - lx) * validf)
            ey = jnp.where(dy, jnp.minimum(y0 + 1, hm1), y0)
            ex = jnp.where(dx, jnp.minimum(x0 + 1, wm1), x0)
            idx_v[cb, pl.ds(pl.multiple_of(jb * _SR, 16), _SR)] = (
                base_i + ey * w_stride + ex)
            w_v[pl.ds(pl.multiple_of(b * _SR, 16), _SR)] = wgt

        lax.fori_loop(0, _NBINS, idx_gen, None)

        def chunk_body(c, _):
            cp = pltpu.async_copy(table.at[idx_v.at[c]], rows_v, sem)
            cp.wait()

            def bin_acc(j, _):
                b = c * _CHUNK_BINS + j
                rbase = j * _SR
                wbase = b * _SR
                ws = [plsc.load_gather(
                    w_v, [jnp.broadcast_to(wbase + rr, (_L,))])
                    for rr in range(_SR)]
                for cc in range(_C // _L):
                    acc = ws[0] * rows_v[rbase, pl.ds(cc * _L, _L)]
                    for rr in range(1, _SR):
                        acc = acc + ws[rr] * rows_v[rbase + rr,
                                                    pl.ds(cc * _L, _L)]
                    plsc.store_scatter(
                        out_v, [lane + cc * _L, jnp.broadcast_to(b, (_L,))],
                        acc)

            lax.fori_loop(0, _CHUNK_BINS, bin_acc, None)

        lax.fori_loop(0, _NCHUNKS, chunk_body, None)
        pltpu.sync_copy(out_v, out.at[r])

    lax.fori_loop(0, _R_PER_W, roi_body, None)


def kernel(feat0, feat1, feat2, feat3, boxes0, boxes1):
    feats = [feat0, feat1, feat2, feat3]
    table = jnp.concatenate(
        [jnp.transpose(f, (0, 2, 3, 1)).reshape(-1, _C) for f in feats],
        axis=0)

    boxes = jnp.concatenate([boxes0, boxes1], axis=0)
    batch = (jnp.arange(_K) // boxes0.shape[0]).astype(jnp.float32)
    cx, cy, w, h, ang = (boxes[:, j] for j in range(5))

    s = jnp.sqrt(w * h)
    target = jnp.floor(
        _CANONICAL_LEVEL + jnp.log2(s / _CANONICAL_SCALE) + 1e-6)
    lvl = (jnp.clip(target, _K_MIN, _K_MAX) - _K_MIN).astype(jnp.int32)
    scale = jnp.take(jnp.array(_SCALES, jnp.float32), lvl)
    hf = jnp.take(jnp.array([float(v) for v in _HS], jnp.float32), lvl)
    off = jnp.take(jnp.array([float(v) for v in _LVL_OFF], jnp.float32), lvl)
    base = off + batch * hf * hf

    cw = cx * scale - 0.5
    chp = cy * scale - 0.5
    rw = w * scale
    rh = h * scale
    theta = ang * (jnp.pi / 180.0)
    params = jnp.stack(
        [cw, chp, rh / _OUT, rw / _OUT, -rh / 2.0, -rw / 2.0,
         jnp.cos(theta), jnp.sin(theta), hf, hf, base, hf,
         jnp.zeros_like(cw), jnp.zeros_like(cw), jnp.zeros_like(cw),
         jnp.zeros_like(cw)], axis=1)

    out = _roi_align_sc(table, params)
    return out.reshape(_K, _C, _OUT, _OUT)


# speedup vs baseline: 36.1926x; 1.2719x over previous
import functools

import jax
import jax.numpy as jnp
from jax import lax
from jax.experimental import pallas as pl
from jax.experimental.pallas import tpu as pltpu
from jax.experimental.pallas import tpu_sc as plsc

_OUT = 7
_NBINS = _OUT * _OUT
_SR = 16
_L = 16
_NC, _NS = 2, 16
_NW = _NC * _NS
_C = 256
_K = 256
_R_PER_W = _K // _NW
_CHUNK_BINS = 7
_CHUNK_ROWS = _CHUNK_BINS * _SR
_NCHUNKS = _NBINS // _CHUNK_BINS

_SCALES = (0.25, 0.125, 0.0625, 0.03125)
_HS = (128, 64, 32, 16)
_LVL_OFF = (0, 32768, 40960, 43008)
_CANONICAL_SCALE = 224.0
_CANONICAL_LEVEL = 4.0
_K_MIN, _K_MAX = 2.0, 5.0


def _make_mesh():
    return plsc.VectorSubcoreMesh(
        core_axis_name="c", subcore_axis_name="s",
        num_cores=_NC, num_subcores=_NS)


@functools.partial(
    pl.kernel,
    out_type=jax.ShapeDtypeStruct((_K, _C, _NBINS), jnp.float32),
    mesh=_make_mesh(),
    scratch_types=[
        pltpu.VMEM((2, _L), jnp.float32),
        pltpu.VMEM((_NCHUNKS, _CHUNK_ROWS), jnp.int32),
        pltpu.VMEM((_NBINS * _SR,), jnp.float32),
        pltpu.VMEM((2, _CHUNK_ROWS, _C), jnp.float32),
        pltpu.VMEM((_C, _NBINS), jnp.float32),
        pltpu.SemaphoreType.DMA,
    ],
    compiler_params=pltpu.CompilerParams(needs_layout_passes=False),
)
def _roi_align_sc(table, params, out, p_v, idx_v, w_v, rows_v, out_v, sem):
    wid = lax.axis_index("s") * _NC + lax.axis_index("c")
    lane = lax.iota(jnp.int32, _L)
    corner = lane & 3
    sub = lane >> 2
    offy = ((sub >> 1).astype(jnp.float32) + 0.5) * 0.5
    offx = ((sub & 1).astype(jnp.float32) + 0.5) * 0.5
    dy = (corner >> 1) == 1
    dx = (corner & 1) == 1

    def roi_body(i, _):
        r = wid * _R_PER_W + i
        pb = i & 1
        pltpu.sync_copy(params.at[r], p_v.at[pb])

        def splat(j):
            return plsc.load_gather(
                p_v, [jnp.broadcast_to(pb, (_L,)),
                      jnp.full((_L,), j, jnp.int32)])

        cw = splat(0)
        ch = splat(1)
        bin_h = splat(2)
        bin_w = splat(3)
        start_h = splat(4)
        start_w = splat(5)
        cos_t = splat(6)
        sin_t = splat(7)
        hf = splat(8)
        wf = splat(9)
        base_i = splat(10).astype(jnp.int32)
        w_stride = splat(11).astype(jnp.int32)
        hm1 = (hf - 1.0).astype(jnp.int32)
        wm1 = (wf - 1.0).astype(jnp.int32)

        def idx_gen(b, _):
            cb = (b * 9363) >> 16
            jb = b - cb * 7
            pyf = cb.astype(jnp.float32)
            pxf = jb.astype(jnp.float32)
            yy = start_h + (pyf + offy) * bin_h
            xx = start_w + (pxf + offx) * bin_w
            y = yy * cos_t - xx * sin_t + ch
            x = yy * sin_t + xx * cos_t + cw
            validf = jnp.where(
                (y > -1.0) & (y < hf) & (x > -1.0) & (x < wf), 0.25, 0.0)
            yc = jnp.clip(y, 0.0, hf - 1.0)
            xc = jnp.clip(x, 0.0, wf - 1.0)
            y0 = yc.astype(jnp.int32)
            x0 = xc.astype(jnp.int32)
            ly = yc - y0.astype(jnp.float32)
            lx = xc - x0.astype(jnp.float32)
            wgt = (jnp.where(dy, ly, 1.0 - ly)
                   * jnp.where(dx, lx, 1.0 - lx) * validf)
            ey = jnp.where(dy, jnp.minimum(y0 + 1, hm1), y0)
            ex = jnp.where(dx, jnp.minimum(x0 + 1, wm1), x0)
            idx_v[cb, pl.ds(pl.multiple_of(jb * _SR, 16), _SR)] = (
                base_i + ey * w_stride + ex)
            w_v[pl.ds(pl.multiple_of(b * _SR, 16), _SR)] = wgt

        lax.fori_loop(0, _NBINS, idx_gen, None)

        pltpu.async_copy(table.at[idx_v.at[0]], rows_v.at[0], sem)

        def chunk_body(c, _):
            par = c & 1
            pltpu.make_async_copy(table.at[idx_v.at[c]],
                                  rows_v.at[par], sem).wait()

            @pl.when(c < _NCHUNKS - 1)
            def _():
                pltpu.async_copy(table.at[idx_v.at[c + 1]],
                                 rows_v.at[1 - par], sem)

            def bin_acc(j, _):
                b = c * _CHUNK_BINS + j
                rbase = j * _SR
                wbase = b * _SR
                ws = [plsc.load_gather(
                    w_v, [jnp.broadcast_to(wbase + rr, (_L,))])
                    for rr in range(_SR)]
                for cc in range(_C // _L):
                    acc = ws[0] * rows_v[par, rbase, pl.ds(cc * _L, _L)]
                    for rr in range(1, _SR):
                        acc = acc + ws[rr] * rows_v[par, rbase + rr,
                                                    pl.ds(cc * _L, _L)]
                    plsc.store_scatter(
                        out_v, [lane + cc * _L, jnp.broadcast_to(b, (_L,))],
                        acc)

            lax.fori_loop(0, _CHUNK_BINS, bin_acc, None)

        lax.fori_loop(0, _NCHUNKS, chunk_body, None)
        pltpu.sync_copy(out_v, out.at[r])

    lax.fori_loop(0, _R_PER_W, roi_body, None)


def kernel(feat0, feat1, feat2, feat3, boxes0, boxes1):
    feats = [feat0, feat1, feat2, feat3]
    table = jnp.concatenate(
        [jnp.transpose(f, (0, 2, 3, 1)).reshape(-1, _C) for f in feats],
        axis=0)

    boxes = jnp.concatenate([boxes0, boxes1], axis=0)
    batch = (jnp.arange(_K) // boxes0.shape[0]).astype(jnp.float32)
    cx, cy, w, h, ang = (boxes[:, j] for j in range(5))

    s = jnp.sqrt(w * h)
    target = jnp.floor(
        _CANONICAL_LEVEL + jnp.log2(s / _CANONICAL_SCALE) + 1e-6)
    lvl = (jnp.clip(target, _K_MIN, _K_MAX) - _K_MIN).astype(jnp.int32)
    scale = jnp.take(jnp.array(_SCALES, jnp.float32), lvl)
    hf = jnp.take(jnp.array([float(v) for v in _HS], jnp.float32), lvl)
    off = jnp.take(jnp.array([float(v) for v in _LVL_OFF], jnp.float32), lvl)
    base = off + batch * hf * hf

    cw = cx * scale - 0.5
    chp = cy * scale - 0.5
    rw = w * scale
    rh = h * scale
    theta = ang * (jnp.pi / 180.0)
    params = jnp.stack(
        [cw, chp, rh / _OUT, rw / _OUT, -rh / 2.0, -rw / 2.0,
         jnp.cos(theta), jnp.sin(theta), hf, hf, base, hf,
         jnp.zeros_like(cw), jnp.zeros_like(cw), jnp.zeros_like(cw),
         jnp.zeros_like(cw)], axis=1)

    out = _roi_align_sc(table, params)
    return out.reshape(_K, _C, _OUT, _OUT)


# speedup vs baseline: 39.9525x; 1.1039x over previous
import functools

import jax
import jax.numpy as jnp
from jax import lax
from jax.experimental import pallas as pl
from jax.experimental.pallas import tpu as pltpu
from jax.experimental.pallas import tpu_sc as plsc

_OUT = 7
_NBINS = _OUT * _OUT
_SR = 16
_L = 16
_NC, _NS = 2, 16
_NW = _NC * _NS
_C = 256
_K = 256
_R_PER_W = _K // _NW
_CHUNK_BINS = 7
_CHUNK_ROWS = _CHUNK_BINS * _SR
_NCHUNKS = _NBINS // _CHUNK_BINS

_SCALES = (0.25, 0.125, 0.0625, 0.03125)
_HS = (128, 64, 32, 16)
_LVL_OFF = (0, 32768, 40960, 43008)
_CANONICAL_SCALE = 224.0
_CANONICAL_LEVEL = 4.0
_K_MIN, _K_MAX = 2.0, 5.0


def _make_mesh():
    return plsc.VectorSubcoreMesh(
        core_axis_name="c", subcore_axis_name="s",
        num_cores=_NC, num_subcores=_NS)


@functools.partial(
    pl.kernel,
    out_type=jax.ShapeDtypeStruct((_K, _C, _NBINS), jnp.float32),
    mesh=_make_mesh(),
    scratch_types=[
        pltpu.VMEM((2, _L), jnp.float32),
        pltpu.VMEM((_NCHUNKS, _CHUNK_ROWS), jnp.int32),
        pltpu.VMEM((_NBINS * _SR,), jnp.float32),
        pltpu.VMEM((2, _CHUNK_ROWS, _C), jnp.float32),
        pltpu.VMEM((_C, _NBINS), jnp.float32),
        pltpu.SemaphoreType.DMA,
    ],
    compiler_params=pltpu.CompilerParams(needs_layout_passes=False),
)
def _roi_align_sc(table, params, out, p_v, idx_v, w_v, rows_v, out_v, sem):
    wid = lax.axis_index("s") * _NC + lax.axis_index("c")
    lane = lax.iota(jnp.int32, _L)
    corner = lane & 3
    sub = lane >> 2
    offy = ((sub >> 1).astype(jnp.float32) + 0.5) * 0.5
    offx = ((sub & 1).astype(jnp.float32) + 0.5) * 0.5
    dy = (corner >> 1) == 1
    dx = (corner & 1) == 1

    def roi_body(i, _):
        r = wid * _R_PER_W + i
        pb = i & 1
        pltpu.sync_copy(params.at[r], p_v.at[pb])

        def splat(j):
            return plsc.load_gather(
                p_v, [jnp.broadcast_to(pb, (_L,)),
                      jnp.full((_L,), j, jnp.int32)])

        cw = splat(0)
        ch = splat(1)
        bin_h = splat(2)
        bin_w = splat(3)
        start_h = splat(4)
        start_w = splat(5)
        cos_t = splat(6)
        sin_t = splat(7)
        hf = splat(8)
        wf = splat(9)
        base_i = splat(10).astype(jnp.int32)
        w_stride = splat(11).astype(jnp.int32)
        hm1 = (hf - 1.0).astype(jnp.int32)
        wm1 = (wf - 1.0).astype(jnp.int32)

        def idx_gen(b, _):
            cb = (b * 9363) >> 16
            jb = b - cb * 7
            pyf = cb.astype(jnp.float32)
            pxf = jb.astype(jnp.float32)
            yy = start_h + (pyf + offy) * bin_h
            xx = start_w + (pxf + offx) * bin_w
            y = yy * cos_t - xx * sin_t + ch
            x = yy * sin_t + xx * cos_t + cw
            validf = jnp.where(
                (y > -1.0) & (y < hf) & (x > -1.0) & (x < wf), 0.25, 0.0)
            yc = jnp.clip(y, 0.0, hf - 1.0)
            xc = jnp.clip(x, 0.0, wf - 1.0)
            y0 = yc.astype(jnp.int32)
            x0 = xc.astype(jnp.int32)
            ly = yc - y0.astype(jnp.float32)
            lx = xc - x0.astype(jnp.float32)
            wgt = (jnp.where(dy, ly, 1.0 - ly)
                   * jnp.where(dx, lx, 1.0 - lx) * validf)
            ey = jnp.where(dy, jnp.minimum(y0 + 1, hm1), y0)
            ex = jnp.where(dx, jnp.minimum(x0 + 1, wm1), x0)
            idx_v[cb, pl.ds(pl.multiple_of(jb * _SR, 16), _SR)] = (
                base_i + ey * w_stride + ex)
            w_v[pl.ds(pl.multiple_of(b * _SR, 16), _SR)] = wgt

        lax.fori_loop(0, _NBINS, idx_gen, None)

        pltpu.async_copy(table.at[idx_v.at[0]], rows_v.at[0], sem)

        def chunk_body(c, _):
            par = c & 1
            pltpu.make_async_copy(table.at[idx_v.at[c]],
                                  rows_v.at[par], sem).wait()

            @pl.when(c < _NCHUNKS - 1)
            def _():
                pltpu.async_copy(table.at[idx_v.at[c + 1]],
                                 rows_v.at[1 - par], sem)

            def bin_acc(j, _):
                b = c * _CHUNK_BINS + j
                rbase = j * _SR
                wbase = pl.multiple_of(b * _SR, 16)
                wvec = w_v[pl.ds(wbase, _SR)]
                ws = [jnp.take_along_axis(
                    wvec, jnp.full((_L,), rr, jnp.int32), axis=0)
                    for rr in range(_SR)]
                for cc in range(_C // _L):
                    vs = [ws[rr] * rows_v[par, rbase + rr, pl.ds(cc * _L, _L)]
                          for rr in range(_SR)]
                    while len(vs) > 1:
                        vs = [vs[t] + vs[t + 1] for t in range(0, len(vs), 2)]
                    plsc.store_scatter(
                        out_v, [lane + cc * _L, jnp.broadcast_to(b, (_L,))],
                        vs[0])

            lax.fori_loop(0, _CHUNK_BINS, bin_acc, None)

        lax.fori_loop(0, _NCHUNKS, chunk_body, None)
        pltpu.sync_copy(out_v, out.at[r])

    lax.fori_loop(0, _R_PER_W, roi_body, None)


def kernel(feat0, feat1, feat2, feat3, boxes0, boxes1):
    feats = [feat0, feat1, feat2, feat3]
    table = jnp.concatenate(
        [jnp.transpose(f, (0, 2, 3, 1)).reshape(-1, _C) for f in feats],
        axis=0)

    boxes = jnp.concatenate([boxes0, boxes1], axis=0)
    batch = (jnp.arange(_K) // boxes0.shape[0]).astype(jnp.float32)
    cx, cy, w, h, ang = (boxes[:, j] for j in range(5))

    s = jnp.sqrt(w * h)
    target = jnp.floor(
        _CANONICAL_LEVEL + jnp.log2(s / _CANONICAL_SCALE) + 1e-6)
    lvl = (jnp.clip(target, _K_MIN, _K_MAX) - _K_MIN).astype(jnp.int32)
    scale = jnp.take(jnp.array(_SCALES, jnp.float32), lvl)
    hf = jnp.take(jnp.array([float(v) for v in _HS], jnp.float32), lvl)
    off = jnp.take(jnp.array([float(v) for v in _LVL_OFF], jnp.float32), lvl)
    base = off + batch * hf * hf

    cw = cx * scale - 0.5
    chp = cy * scale - 0.5
    rw = w * scale
    rh = h * scale
    theta = ang * (jnp.pi / 180.0)
    params = jnp.stack(
        [cw, chp, rh / _OUT, rw / _OUT, -rh / 2.0, -rw / 2.0,
         jnp.cos(theta), jnp.sin(theta), hf, hf, base, hf,
         jnp.zeros_like(cw), jnp.zeros_like(cw), jnp.zeros_like(cw),
         jnp.zeros_like(cw)], axis=1)

    out = _roi_align_sc(table, params)
    return out.reshape(_K, _C, _OUT, _OUT)


# speedup vs baseline: 42.4415x; 1.0623x over previous
import functools

import jax
import jax.numpy as jnp
from jax import lax
from jax.experimental import pallas as pl
from jax.experimental.pallas import tpu as pltpu
from jax.experimental.pallas import tpu_sc as plsc

_OUT = 7
_NBINS = _OUT * _OUT
_SR = 16
_L = 16
_NC, _NS = 2, 16
_NW = _NC * _NS
_C = 256
_K = 256
_R_PER_W = _K // _NW
_CHUNK_BINS = 7
_CHUNK_ROWS = _CHUNK_BINS * _SR
_NCHUNKS = _NBINS // _CHUNK_BINS

_SCALES = (0.25, 0.125, 0.0625, 0.03125)
_HS = (128, 64, 32, 16)
_LVL_OFF = (0, 32768, 40960, 43008)
_CANONICAL_SCALE = 224.0
_CANONICAL_LEVEL = 4.0
_K_MIN, _K_MAX = 2.0, 5.0


def _make_mesh():
    return plsc.VectorSubcoreMesh(
        core_axis_name="c", subcore_axis_name="s",
        num_cores=_NC, num_subcores=_NS)


@functools.partial(
    pl.kernel,
    out_type=jax.ShapeDtypeStruct((_K, _C, _NBINS), jnp.float32),
    mesh=_make_mesh(),
    scratch_types=[
        pltpu.VMEM((2, _L), jnp.float32),
        pltpu.VMEM((_NCHUNKS, _CHUNK_ROWS), jnp.int32),
        pltpu.VMEM((_NBINS * _SR,), jnp.float32),
        pltpu.VMEM((2, _CHUNK_ROWS, _C), jnp.float32),
        pltpu.VMEM((_C, _NBINS), jnp.float32),
        pltpu.SemaphoreType.DMA,
    ],
    compiler_params=pltpu.CompilerParams(needs_layout_passes=False),
)
def _roi_align_sc(table, params, out, p_v, idx_v, w_v, rows_v, out_v, sem):
    wid = lax.axis_index("s") * _NC + lax.axis_index("c")
    lane = lax.iota(jnp.int32, _L)
    corner = lane & 3
    sub = lane >> 2
    offy = ((sub >> 1).astype(jnp.float32) + 0.5) * 0.5
    offx = ((sub & 1).astype(jnp.float32) + 0.5) * 0.5
    dy = (corner >> 1) == 1
    dx = (corner & 1) == 1

    def roi_body(i, _):
        r = wid * _R_PER_W + i
        pb = i & 1
        pltpu.sync_copy(params.at[r], p_v.at[pb])

        def splat(j):
            return plsc.load_gather(
                p_v, [jnp.broadcast_to(pb, (_L,)),
                      jnp.full((_L,), j, jnp.int32)])

        cw = splat(0)
        ch = splat(1)
        bin_h = splat(2)
        bin_w = splat(3)
        start_h = splat(4)
        start_w = splat(5)
        cos_t = splat(6)
        sin_t = splat(7)
        hf = splat(8)
        wf = splat(9)
        base_i = splat(10).astype(jnp.int32)
        w_stride = splat(11).astype(jnp.int32)
        hm1 = (hf - 1.0).astype(jnp.int32)
        wm1 = (wf - 1.0).astype(jnp.int32)

        @plsc.parallel_loop(0, _NBINS, unroll=2)
        def idx_gen(b):
            cb = (b * 9363) >> 16
            jb = b - cb * 7
            pyf = cb.astype(jnp.float32)
            pxf = jb.astype(jnp.float32)
            yy = start_h + (pyf + offy) * bin_h
            xx = start_w + (pxf + offx) * bin_w
            y = yy * cos_t - xx * sin_t + ch
            x = yy * sin_t + xx * cos_t + cw
            validf = jnp.where(
                (y > -1.0) & (y < hf) & (x > -1.0) & (x < wf), 0.25, 0.0)
            yc = jnp.clip(y, 0.0, hf - 1.0)
            xc = jnp.clip(x, 0.0, wf - 1.0)
            y0 = yc.astype(jnp.int32)
            x0 = xc.astype(jnp.int32)
            ly = yc - y0.astype(jnp.float32)
            lx = xc - x0.astype(jnp.float32)
            wgt = (jnp.where(dy, ly, 1.0 - ly)
                   * jnp.where(dx, lx, 1.0 - lx) * validf)
            ey = jnp.where(dy, jnp.minimum(y0 + 1, hm1), y0)
            ex = jnp.where(dx, jnp.minimum(x0 + 1, wm1), x0)
            idx_v[cb, pl.ds(pl.multiple_of(jb * _SR, 16), _SR)] = (
                base_i + ey * w_stride + ex)
            w_v[pl.ds(pl.multiple_of(b * _SR, 16), _SR)] = wgt

        pltpu.async_copy(table.at[idx_v.at[0]], rows_v.at[0], sem)

        def chunk_body(c, _):
            par = c & 1
            pltpu.make_async_copy(table.at[idx_v.at[c]],
                                  rows_v.at[par], sem).wait()

            @pl.when(c < _NCHUNKS - 1)
            def _():
                pltpu.async_copy(table.at[idx_v.at[c + 1]],
                                 rows_v.at[1 - par], sem)

            @plsc.parallel_loop(0, _CHUNK_BINS, unroll=2)
            def bin_acc(j):
                b = c * _CHUNK_BINS + j
                rbase = j * _SR
                wbase = pl.multiple_of(b * _SR, 16)
                wvec = w_v[pl.ds(wbase, _SR)]
                ws = [jnp.take_along_axis(
                    wvec, jnp.full((_L,), rr, jnp.int32), axis=0)
                    for rr in range(_SR)]
                for cc in range(_C // _L):
                    vs = [ws[rr] * rows_v[par, rbase + rr, pl.ds(cc * _L, _L)]
                          for rr in range(_SR)]
                    while len(vs) > 1:
                        vs = [vs[t] + vs[t + 1] for t in range(0, len(vs), 2)]
                    plsc.store_scatter(
                        out_v, [lane + cc * _L, jnp.broadcast_to(b, (_L,))],
                        vs[0])

        lax.fori_loop(0, _NCHUNKS, chunk_body, None)
        pltpu.sync_copy(out_v, out.at[r])

    lax.fori_loop(0, _R_PER_W, roi_body, None)


def kernel(feat0, feat1, feat2, feat3, boxes0, boxes1):
    feats = [feat0, feat1, feat2, feat3]
    table = jnp.concatenate(
        [jnp.transpose(f, (0, 2, 3, 1)).reshape(-1, _C) for f in feats],
        axis=0)

    boxes = jnp.concatenate([boxes0, boxes1], axis=0)
    batch = (jnp.arange(_K) // boxes0.shape[0]).astype(jnp.float32)
    cx, cy, w, h, ang = (boxes[:, j] for j in range(5))

    s = jnp.sqrt(w * h)
    target = jnp.floor(
        _CANONICAL_LEVEL + jnp.log2(s / _CANONICAL_SCALE) + 1e-6)
    lvl = (jnp.clip(target, _K_MIN, _K_MAX) - _K_MIN).astype(jnp.int32)
    scale = jnp.take(jnp.array(_SCALES, jnp.float32), lvl)
    hf = jnp.take(jnp.array([float(v) for v in _HS], jnp.float32), lvl)
    off = jnp.take(jnp.array([float(v) for v in _LVL_OFF], jnp.float32), lvl)
    base = off + batch * hf * hf

    cw = cx * scale - 0.5
    chp = cy * scale - 0.5
    rw = w * scale
    rh = h * scale
    theta = ang * (jnp.pi / 180.0)
    params = jnp.stack(
        [cw, chp, rh / _OUT, rw / _OUT, -rh / 2.0, -rw / 2.0,
         jnp.cos(theta), jnp.sin(theta), hf, hf, base, hf,
         jnp.zeros_like(cw), jnp.zeros_like(cw), jnp.zeros_like(cw),
         jnp.zeros_like(cw)], axis=1)

    out = _roi_align_sc(table, params)
    return out.reshape(_K, _C, _OUT, _OUT)
